# Initial kernel scaffold; baseline (speedup 1.0000x reference)
import jax
import jax.numpy as jnp
from jax.experimental import pallas as pl


def kernel(codes, map_a, map_b, W):
    raise NotImplementedError("write your pallas kernel here")


# speedup vs baseline: 41.9962x; 41.9962x over previous
import functools

import jax
import jax.numpy as jnp
from jax import lax
from jax.experimental import pallas as pl
from jax.experimental.pallas import tpu as pltpu
from jax.experimental.pallas import tpu_sc as plsc

_SIZES = (65536, 262144)
_R = 1048576
_HDIM = 4
_OUT = 32

_NC, _NS = 2, 16
_NW = _NC * _NS
_RPW = _R // _NW
_CB = 4096
_G = _CB // 128
_NCHUNK = _RPW // _CB
_GP128 = _R // 128


def _sc_gather(ta0, ta1, ta2, ta3, tb0, tb1, tb2, tb3, ma3, mb3):
    mesh = plsc.VectorSubcoreMesh(core_axis_name="c", subcore_axis_name="s")

    @functools.partial(
        pl.kernel,
        mesh=mesh,
        out_type=jax.ShapeDtypeStruct((8, _GP128, 128), jnp.float32),
        scratch_types=[
            pltpu.VMEM((8, _G, 128), jnp.int32),
            pltpu.VMEM((8, _G, 128), jnp.float32),
            pltpu.SemaphoreType.DMA,
        ],
    )
    def k(a0, a1, a2, a3, b0, b1, b2, b3, ma, mb, feat_hbm, idx_v, g_v, sem):
        tables = (a0, a1, a2, a3, b0, b1, b2, b3)
        wid = lax.axis_index("s") * _NC + lax.axis_index("c")

        def chunk(ci, carry):
            gbase = wid * (_RPW // 128) + ci * _G
            pltpu.sync_copy(ma.at[:, pl.ds(gbase, _G)], idx_v.at[pl.ds(0, 4)])
            pltpu.sync_copy(mb.at[:, pl.ds(gbase, _G)], idx_v.at[pl.ds(4, 4)])

            def grp(g, c2):
                copies = [
                    pltpu.async_copy(tables[j].at[idx_v.at[j, g]],
                                     g_v.at[j, g], sem)
                    for j in range(8)
                ]
                for cp in copies:
                    cp.wait()
                return c2

            lax.fori_loop(0, _G, grp, 0)
            pltpu.sync_copy(g_v, feat_hbm.at[:, pl.ds(gbase, _G)])
            return carry

        lax.fori_loop(0, _NCHUNK, chunk, 0)

    return k(ta0, ta1, ta2, ta3, tb0, tb1, tb2, tb3, ma3, mb3)


def _mm_body(ft_ref, w_ref, o_ref):
    o_ref[...] = lax.dot_general(
        ft_ref[...], w_ref[...], (((0,), (0,)), ((), ())),
        preferred_element_type=jnp.float32)


def _mm(ft, W):
    br = 2048
    return pl.pallas_call(
        _mm_body,
        grid=(_R // br,),
        in_specs=[
            pl.BlockSpec((8, br), lambda i: (0, i)),
            pl.BlockSpec((8, _OUT), lambda i: (0, 0)),
        ],
        out_specs=pl.BlockSpec((br, _OUT), lambda i: (i, 0)),
        out_shape=jax.ShapeDtypeStruct((_R, _OUT), jnp.float32),
    )(ft, W)


def kernel(codes, map_a, map_b, W):
    ca = codes[:_SIZES[0]].T
    cb = codes[_SIZES[0]:].T
    ma3 = map_a.T.reshape(_HDIM, _GP128, 128)
    mb3 = map_b.T.reshape(_HDIM, _GP128, 128)
    feat3 = _sc_gather(ca[0], ca[1], ca[2], ca[3],
                       cb[0], cb[1], cb[2], cb[3], ma3, mb3)
    ft = feat3.reshape(8, _R)
    return _mm(ft, W)


# speedup vs baseline: 57.7615x; 1.3754x over previous
import functools

import jax
import jax.numpy as jnp
from jax import lax
from jax.experimental import pallas as pl
from jax.experimental.pallas import tpu as pltpu
from jax.experimental.pallas import tpu_sc as plsc

_SIZES = (65536, 262144)
_R = 1048576
_HDIM = 4
_OUT = 32

_NC, _NS = 2, 16
_NW = _NC * _NS
_RPW = _R // _NW
_CB = 2048
_G = _CB // 128
_NCHUNK = _RPW // _CB
_GP128 = _R // 128


def _sc_gather(ta0, ta1, ta2, ta3, tb0, tb1, tb2, tb3, ma3, mb3):
    mesh = plsc.VectorSubcoreMesh(core_axis_name="c", subcore_axis_name="s")

    @functools.partial(
        pl.kernel,
        mesh=mesh,
        out_type=jax.ShapeDtypeStruct((8, _GP128, 128), jnp.float32),
        scratch_types=[
            pltpu.VMEM((8, _G, 128), jnp.int32),
            pltpu.VMEM((8, _G, 128), jnp.float32),
            pltpu.SemaphoreType.DMA,
            pltpu.VMEM_SHARED((_SIZES[0],), jnp.float32),
            pltpu.VMEM_SHARED((_SIZES[0],), jnp.float32),
            pltpu.VMEM_SHARED((_SIZES[0],), jnp.float32),
            pltpu.VMEM_SHARED((_SIZES[0],), jnp.float32),
            pltpu.VMEM_SHARED((_SIZES[1],), jnp.float32),
            pltpu.VMEM_SHARED((_SIZES[1],), jnp.float32),
            pltpu.VMEM_SHARED((_SIZES[1],), jnp.float32),
            pltpu.VMEM_SHARED((_SIZES[1],), jnp.float32),
        ],
    )
    def k(a0, a1, a2, a3, b0, b1, b2, b3, ma, mb, feat_hbm, idx_v, g_v, sem,
          s0, s1, s2, s3, s4, s5, s6, s7):
        hbm_tables = (a0, a1, a2, a3, b0, b1, b2, b3)
        tables = (s0, s1, s2, s3, s4, s5, s6, s7)
        sid = lax.axis_index("s")
        wid = sid * _NC + lax.axis_index("c")

        for j in range(8):
            @pl.when(sid == j)
            def _():
                pltpu.sync_copy(hbm_tables[j], tables[j])
        plsc.subcore_barrier()

        def chunk(ci, carry):
            gbase = wid * (_RPW // 128) + ci * _G
            pltpu.sync_copy(ma.at[:, pl.ds(gbase, _G)], idx_v.at[pl.ds(0, 4)])
            pltpu.sync_copy(mb.at[:, pl.ds(gbase, _G)], idx_v.at[pl.ds(4, 4)])

            def grp(g, c2):
                copies = [
                    pltpu.async_copy(tables[j].at[idx_v.at[j, g]],
                                     g_v.at[j, g], sem)
                    for j in range(8)
                ]
                for cp in copies:
                    cp.wait()
                return c2

            lax.fori_loop(0, _G, grp, 0)
            pltpu.sync_copy(g_v, feat_hbm.at[:, pl.ds(gbase, _G)])
            return carry

        lax.fori_loop(0, _NCHUNK, chunk, 0)

    return k(ta0, ta1, ta2, ta3, tb0, tb1, tb2, tb3, ma3, mb3)


def _mm_body(ft_ref, w_ref, o_ref):
    o_ref[...] = lax.dot_general(
        ft_ref[...], w_ref[...], (((0,), (0,)), ((), ())),
        preferred_element_type=jnp.float32)


def _mm(ft, W):
    br = 2048
    return pl.pallas_call(
        _mm_body,
        grid=(_R // br,),
        in_specs=[
            pl.BlockSpec((8, br), lambda i: (0, i)),
            pl.BlockSpec((8, _OUT), lambda i: (0, 0)),
        ],
        out_specs=pl.BlockSpec((br, _OUT), lambda i: (i, 0)),
        out_shape=jax.ShapeDtypeStruct((_R, _OUT), jnp.float32),
    )(ft, W)


def kernel(codes, map_a, map_b, W):
    ca = codes[:_SIZES[0]].T
    cb = codes[_SIZES[0]:].T
    ma3 = map_a.T.reshape(_HDIM, _GP128, 128)
    mb3 = map_b.T.reshape(_HDIM, _GP128, 128)
    feat3 = _sc_gather(ca[0], ca[1], ca[2], ca[3],
                       cb[0], cb[1], cb[2], cb[3], ma3, mb3)
    ft = feat3.reshape(8, _R)
    return _mm(ft, W)


# speedup vs baseline: 96.0104x; 1.6622x over previous
import functools

import jax
import jax.numpy as jnp
from jax import lax
from jax.experimental import pallas as pl
from jax.experimental.pallas import tpu as pltpu
from jax.experimental.pallas import tpu_sc as plsc

_SIZES = (65536, 262144)
_R = 1048576
_HDIM = 4
_OUT = 32
_TW = (_SIZES[0] + _SIZES[1]) * _HDIM

_NC, _NS = 2, 16
_NW = _NC * _NS
_RPW = _R // _NW
_CB = 1024
_G = _CB // 128
_NCHUNK = _RPW // _CB
_GP128 = _R // 128
_STW = _TW // _NS


def _sc_gather(codes_flat, map_a, map_b):
    mesh = plsc.VectorSubcoreMesh(core_axis_name="c", subcore_axis_name="s")

    @functools.partial(
        pl.kernel,
        mesh=mesh,
        out_type=jax.ShapeDtypeStruct((8, _GP128, 128), jnp.float32),
        scratch_types=[
            pltpu.VMEM((8, _G, 128), jnp.int32),
            pltpu.VMEM((8, _G, 128), jnp.int32),
            pltpu.VMEM((8, _G, 128), jnp.float32),
            pltpu.SemaphoreType.DMA,
            pltpu.VMEM_SHARED((_SIZES[1] * _HDIM,), jnp.float32),
            pltpu.VMEM_SHARED((_SIZES[0] * _HDIM,), jnp.float32),
        ],
    )
    def k(cb_hbm, ca_hbm, ma_hbm, mb_hbm, feat_hbm, mval, idx_v, g_v,
          sem, tb, ta):
        sid = lax.axis_index("s")
        wid = sid * _NC + lax.axis_index("c")
        iota = lax.iota(jnp.int32, 16)

        wb = _SIZES[1] * _HDIM // _NS
        wa = _SIZES[0] * _HDIM // _NS
        pltpu.sync_copy(cb_hbm.at[pl.ds(sid * wb, wb)],
                        tb.at[pl.ds(sid * wb, wb)])
        pltpu.sync_copy(ca_hbm.at[pl.ds(sid * wa, wa)],
                        ta.at[pl.ds(sid * wa, wa)])
        plsc.subcore_barrier()

        def chunk(ci, carry):
            gbase = wid * (_RPW // 128) + ci * _G
            pltpu.sync_copy(ma_hbm.at[:, pl.ds(gbase, _G)], mval.at[pl.ds(0, 4)])
            pltpu.sync_copy(mb_hbm.at[:, pl.ds(gbase, _G)], mval.at[pl.ds(4, 4)])

            for j in range(8):
                i = j % 4

                def sub(g, c2):
                    for t8 in range(8):
                        m = mval[j, g, pl.ds(t8 * 16, 16)]
                        idx_v[j, g, pl.ds(t8 * 16, 16)] = m * 4 + i
                    return c2

                lax.fori_loop(0, _G, sub, 0)

            def grp(g, c2):
                copies = [
                    pltpu.async_copy((ta if j < 4 else tb).at[idx_v.at[j, g]],
                                     g_v.at[j, g], sem)
                    for j in range(8)
                ]
                for cp in copies:
                    cp.wait()
                return c2

            lax.fori_loop(0, _G, grp, 0)
            pltpu.sync_copy(g_v, feat_hbm.at[:, pl.ds(gbase, _G)])
            return carry

        lax.fori_loop(0, _NCHUNK, chunk, 0)

    return k(codes_flat[0], codes_flat[1], map_a, map_b)


def _mm_body(w_ref, ft_ref, o_ref):
    o_ref[...] = lax.dot_general(
        w_ref[...], ft_ref[...], (((0,), (0,)), ((), ())),
        preferred_element_type=jnp.float32)


def _mm_t(ft, W):
    br = 8192
    return pl.pallas_call(
        _mm_body,
        grid=(_R // br,),
        in_specs=[
            pl.BlockSpec((8, _OUT), lambda i: (0, 0)),
            pl.BlockSpec((8, br), lambda i: (0, i)),
        ],
        out_specs=pl.BlockSpec((_OUT, br), lambda i: (0, i)),
        out_shape=jax.ShapeDtypeStruct((_OUT, _R), jnp.float32),
    )(W, ft)


def kernel(codes, map_a, map_b, W):
    cbf = codes[_SIZES[0]:].reshape(-1)
    caf = codes[:_SIZES[0]].reshape(-1)
    feat3 = _sc_gather((cbf, caf),
                       map_a.T.reshape(_HDIM, _GP128, 128),
                       map_b.T.reshape(_HDIM, _GP128, 128))
    ft = feat3.reshape(8, _R)
    return _mm_t(ft, W).T


# speedup vs baseline: 178.2233x; 1.8563x over previous
import functools

import jax
import jax.numpy as jnp
from jax import lax
from jax.experimental import pallas as pl
from jax.experimental.pallas import tpu as pltpu
from jax.experimental.pallas import tpu_sc as plsc

_SIZES = (65536, 262144)
_R = 1048576
_HDIM = 4
_OUT = 32

_NC, _NS = 2, 16
_NW = _NC * _NS
_RPW = _R // _NW
_CB = 2048
_G = _CB // 128
_NCHUNK = _RPW // _CB
_GP128 = _R // 128


def _sc_gather(ta0, ta1, ta2, ta3, tb0, tb1, tb2, tb3, ma3, mb3):
    mesh = plsc.VectorSubcoreMesh(core_axis_name="c", subcore_axis_name="s")

    @functools.partial(
        pl.kernel,
        mesh=mesh,
        out_type=jax.ShapeDtypeStruct((8, _GP128, 128), jnp.float32),
        scratch_types=[
            pltpu.VMEM((8, _G, 128), jnp.int32),
            pltpu.VMEM((8, _G, 128), jnp.float32),
            pltpu.SemaphoreType.DMA,
            pltpu.VMEM_SHARED((_SIZES[0],), jnp.float32),
            pltpu.VMEM_SHARED((_SIZES[0],), jnp.float32),
            pltpu.VMEM_SHARED((_SIZES[0],), jnp.float32),
            pltpu.VMEM_SHARED((_SIZES[0],), jnp.float32),
            pltpu.VMEM_SHARED((_SIZES[1],), jnp.float32),
            pltpu.VMEM_SHARED((_SIZES[1],), jnp.float32),
            pltpu.VMEM_SHARED((_SIZES[1],), jnp.float32),
            pltpu.VMEM_SHARED((_SIZES[1],), jnp.float32),
        ],
    )
    def k(a0, a1, a2, a3, b0, b1, b2, b3, ma, mb, feat_hbm, idx_v, g_v, sem,
          s0, s1, s2, s3, s4, s5, s6, s7):
        hbm_tables = (a0, a1, a2, a3, b0, b1, b2, b3)
        tables = (s0, s1, s2, s3, s4, s5, s6, s7)
        sid = lax.axis_index("s")
        wid = sid * _NC + lax.axis_index("c")

        for j in range(8):
            @pl.when(sid == j)
            def _():
                pltpu.sync_copy(hbm_tables[j], tables[j])
        plsc.subcore_barrier()

        def chunk(ci, carry):
            gbase = wid * (_RPW // 128) + ci * _G
            pltpu.sync_copy(ma.at[:, pl.ds(gbase, _G)], idx_v.at[pl.ds(0, 4)])
            pltpu.sync_copy(mb.at[:, pl.ds(gbase, _G)], idx_v.at[pl.ds(4, 4)])

            def grp(g, c2):
                copies = [
                    pltpu.async_copy(tables[j].at[idx_v.at[j, g]],
                                     g_v.at[j, g], sem)
                    for j in range(8)
                ]
                for cp in copies:
                    cp.wait()
                return c2

            lax.fori_loop(0, _G, grp, 0)
            pltpu.sync_copy(g_v, feat_hbm.at[:, pl.ds(gbase, _G)])
            return carry

        lax.fori_loop(0, _NCHUNK, chunk, 0)

    return k(ta0, ta1, ta2, ta3, tb0, tb1, tb2, tb3, ma3, mb3)


_GBLK = 64
_BR = _GBLK * 128


def _mm_body(w_ref, ft_ref, o_ref):
    ft = ft_ref[...].reshape(8, _BR)
    o_ref[...] = lax.dot_general(
        w_ref[...], ft, (((0,), (0,)), ((), ())),
        preferred_element_type=jnp.float32)


def _mm_t(feat3, W):
    return pl.pallas_call(
        _mm_body,
        grid=(_R // _BR,),
        in_specs=[
            pl.BlockSpec((8, _OUT), lambda i: (0, 0)),
            pl.BlockSpec((8, _GBLK, 128), lambda i: (0, i, 0)),
        ],
        out_specs=pl.BlockSpec((_OUT, _BR), lambda i: (0, i)),
        out_shape=jax.ShapeDtypeStruct((_OUT, _R), jnp.float32),
    )(W, feat3)


def kernel(codes, map_a, map_b, W):
    ca = codes[:_SIZES[0]].T
    cb = codes[_SIZES[0]:].T
    ma3 = map_a.T.reshape(_HDIM, _GP128, 128)
    mb3 = map_b.T.reshape(_HDIM, _GP128, 128)
    feat3 = _sc_gather(ca[0], ca[1], ca[2], ca[3],
                       cb[0], cb[1], cb[2], cb[3], ma3, mb3)
    return _mm_t(feat3, W).T


# speedup vs baseline: 236.2870x; 1.3258x over previous
import functools

import jax
import jax.numpy as jnp
from jax import lax
from jax.experimental import pallas as pl
from jax.experimental.pallas import tpu as pltpu
from jax.experimental.pallas import tpu_sc as plsc

_SIZES = (65536, 262144)
_R = 1048576
_HDIM = 4
_OUT = 32

_NC, _NS = 2, 16
_NW = _NC * _NS
_RPW = _R // _NW
_CB = 1024
_G = _CB // 128
_NCHUNK = _RPW // _CB
_GP128 = _R // 128


def _sc_gather(ta0, ta1, ta2, ta3, tb0, tb1, tb2, tb3, ma3, mb3):
    mesh = plsc.VectorSubcoreMesh(core_axis_name="c", subcore_axis_name="s")

    @functools.partial(
        pl.kernel,
        mesh=mesh,
        out_type=jax.ShapeDtypeStruct((8, _GP128, 128), jnp.float32),
        scratch_types=[
            pltpu.VMEM((2, 8, _G, 128), jnp.int32),
            pltpu.VMEM((2, 8, _G, 128), jnp.float32),
            pltpu.SemaphoreType.DMA,
            pltpu.SemaphoreType.DMA,
            pltpu.SemaphoreType.DMA,
            pltpu.VMEM_SHARED((_SIZES[0],), jnp.float32),
            pltpu.VMEM_SHARED((_SIZES[0],), jnp.float32),
            pltpu.VMEM_SHARED((_SIZES[0],), jnp.float32),
            pltpu.VMEM_SHARED((_SIZES[0],), jnp.float32),
            pltpu.VMEM_SHARED((_SIZES[1],), jnp.float32),
            pltpu.VMEM_SHARED((_SIZES[1],), jnp.float32),
            pltpu.VMEM_SHARED((_SIZES[1],), jnp.float32),
            pltpu.VMEM_SHARED((_SIZES[1],), jnp.float32),
        ],
    )
    def k(a0, a1, a2, a3, b0, b1, b2, b3, ma, mb, feat_hbm, idx_v, g_v, sem,
          msem, wsem, s0, s1, s2, s3, s4, s5, s6, s7):
        hbm_tables = (a0, a1, a2, a3, b0, b1, b2, b3)
        tables = (s0, s1, s2, s3, s4, s5, s6, s7)
        sid = lax.axis_index("s")
        wid = sid * _NC + lax.axis_index("c")

        for j in range(8):
            @pl.when(sid == j)
            def _():
                pltpu.sync_copy(hbm_tables[j], tables[j])
        plsc.subcore_barrier()

        gb0 = wid * (_RPW // 128)

        def start_maps(ci, buf):
            gbase = gb0 + ci * _G
            pltpu.async_copy(ma.at[:, pl.ds(gbase, _G)],
                             idx_v.at[buf, pl.ds(0, 4)], msem)
            pltpu.async_copy(mb.at[:, pl.ds(gbase, _G)],
                             idx_v.at[buf, pl.ds(4, 4)], msem)

        def drain_maps(buf):
            pltpu.make_async_copy(ma.at[:, pl.ds(0, _G)],
                                  idx_v.at[buf, pl.ds(0, 4)], msem).wait()
            pltpu.make_async_copy(mb.at[:, pl.ds(0, _G)],
                                  idx_v.at[buf, pl.ds(4, 4)], msem).wait()

        def drain_feat(buf):
            pltpu.make_async_copy(g_v.at[buf],
                                  feat_hbm.at[:, pl.ds(0, _G)], wsem).wait()

        start_maps(0, 0)

        def chunk(ci, carry):
            buf = lax.rem(ci, 2)
            nbuf = lax.rem(ci + 1, 2)

            @pl.when(ci + 1 < _NCHUNK)
            def _():
                start_maps(ci + 1, nbuf)

            drain_maps(buf)

            @pl.when(ci >= 2)
            def _():
                drain_feat(buf)

            def grp(g, c2):
                copies = [
                    pltpu.async_copy(tables[j].at[idx_v.at[buf, j, g]],
                                     g_v.at[buf, j, g], sem)
                    for j in range(8)
                ]
                for cp in copies:
                    cp.wait()
                return c2

            lax.fori_loop(0, _G, grp, 0)
            gbase = gb0 + ci * _G
            pltpu.async_copy(g_v.at[buf], feat_hbm.at[:, pl.ds(gbase, _G)],
                             wsem)
            return carry

        lax.fori_loop(0, _NCHUNK, chunk, 0)
        drain_feat(0)
        drain_feat(1)

    return k(ta0, ta1, ta2, ta3, tb0, tb1, tb2, tb3, ma3, mb3)


_GBLK = 128
_BR = _GBLK * 128


def _mm_body(w_ref, ft_ref, o_ref):
    ft = ft_ref[...].reshape(8, _BR)
    o_ref[...] = lax.dot_general(
        w_ref[...], ft, (((0,), (0,)), ((), ())),
        preferred_element_type=jnp.float32)


def _mm_t(feat3, W):
    return pl.pallas_call(
        _mm_body,
        grid=(_R // _BR,),
        in_specs=[
            pl.BlockSpec((8, _OUT), lambda i: (0, 0)),
            pl.BlockSpec((8, _GBLK, 128), lambda i: (0, i, 0)),
        ],
        out_specs=pl.BlockSpec((_OUT, _BR), lambda i: (0, i)),
        out_shape=jax.ShapeDtypeStruct((_OUT, _R), jnp.float32),
    )(W, feat3)


def kernel(codes, map_a, map_b, W):
    ca = codes[:_SIZES[0]].T
    cb = codes[_SIZES[0]:].T
    ma3 = map_a.T.reshape(_HDIM, _GP128, 128)
    mb3 = map_b.T.reshape(_HDIM, _GP128, 128)
    feat3 = _sc_gather(ca[0], ca[1], ca[2], ca[3],
                       cb[0], cb[1], cb[2], cb[3], ma3, mb3)
    return _mm_t(feat3, W).T


# speedup vs baseline: 263.3916x; 1.1147x over previous
import functools

import jax
import jax.numpy as jnp
from jax import lax
from jax.experimental import pallas as pl
from jax.experimental.pallas import tpu as pltpu
from jax.experimental.pallas import tpu_sc as plsc

_SIZES = (65536, 262144)
_R = 1048576
_HDIM = 4
_OUT = 32

_NC, _NS = 2, 16
_NW = _NC * _NS
_RPW = _R // _NW
_CB = 1024
_G = _CB // 128
_NCHUNK = _RPW // _CB
_GP128 = _R // 128
_HALF_G = _GP128 // 2
_RPWH = _RPW // 2
_NCHUNKH = _RPWH // _CB


def _sc_gather(half, ta0, ta1, ta2, ta3, tb0, tb1, tb2, tb3, ma3, mb3):
    mesh = plsc.VectorSubcoreMesh(core_axis_name="c", subcore_axis_name="s")

    @functools.partial(
        pl.kernel,
        mesh=mesh,
        out_type=jax.ShapeDtypeStruct((8, _HALF_G, 128), jnp.float32),
        scratch_types=[
            pltpu.VMEM((2, 8, _G, 128), jnp.int32),
            pltpu.VMEM((2, 8, _G, 128), jnp.float32),
            pltpu.SemaphoreType.DMA,
            pltpu.SemaphoreType.DMA,
            pltpu.SemaphoreType.DMA,
            pltpu.VMEM_SHARED((_SIZES[0],), jnp.float32),
            pltpu.VMEM_SHARED((_SIZES[0],), jnp.float32),
            pltpu.VMEM_SHARED((_SIZES[0],), jnp.float32),
            pltpu.VMEM_SHARED((_SIZES[0],), jnp.float32),
            pltpu.VMEM_SHARED((_SIZES[1],), jnp.float32),
            pltpu.VMEM_SHARED((_SIZES[1],), jnp.float32),
            pltpu.VMEM_SHARED((_SIZES[1],), jnp.float32),
            pltpu.VMEM_SHARED((_SIZES[1],), jnp.float32),
        ],
    )
    def k(a0, a1, a2, a3, b0, b1, b2, b3, ma, mb, feat_hbm, idx_v, g_v, sem,
          msem, wsem, s0, s1, s2, s3, s4, s5, s6, s7):
        hbm_tables = (a0, a1, a2, a3, b0, b1, b2, b3)
        tables = (s0, s1, s2, s3, s4, s5, s6, s7)
        sid = lax.axis_index("s")
        wid = sid * _NC + lax.axis_index("c")

        for j in range(8):
            @pl.when(sid == j)
            def _():
                pltpu.sync_copy(hbm_tables[j], tables[j])
        plsc.subcore_barrier()

        gb0 = wid * (_RPWH // 128)
        gsrc0 = half * _HALF_G + gb0

        def start_maps(ci, buf):
            gbase = gsrc0 + ci * _G
            pltpu.async_copy(ma.at[:, pl.ds(gbase, _G)],
                             idx_v.at[buf, pl.ds(0, 4)], msem)
            pltpu.async_copy(mb.at[:, pl.ds(gbase, _G)],
                             idx_v.at[buf, pl.ds(4, 4)], msem)

        def drain_maps(buf):
            pltpu.make_async_copy(ma.at[:, pl.ds(0, _G)],
                                  idx_v.at[buf, pl.ds(0, 4)], msem).wait()
            pltpu.make_async_copy(mb.at[:, pl.ds(0, _G)],
                                  idx_v.at[buf, pl.ds(4, 4)], msem).wait()

        def drain_feat(buf):
            pltpu.make_async_copy(g_v.at[buf],
                                  feat_hbm.at[:, pl.ds(0, _G)], wsem).wait()

        start_maps(0, 0)

        def chunk(ci, carry):
            buf = lax.rem(ci, 2)
            nbuf = lax.rem(ci + 1, 2)

            @pl.when(ci + 1 < _NCHUNKH)
            def _():
                start_maps(ci + 1, nbuf)

            drain_maps(buf)

            @pl.when(ci >= 2)
            def _():
                drain_feat(buf)

            def grp(g, c2):
                copies = [
                    pltpu.async_copy(tables[j].at[idx_v.at[buf, j, g]],
                                     g_v.at[buf, j, g], sem)
                    for j in range(8)
                ]
                for cp in copies:
                    cp.wait()
                return c2

            lax.fori_loop(0, _G, grp, 0)
            gbase = gb0 + ci * _G
            pltpu.async_copy(g_v.at[buf], feat_hbm.at[:, pl.ds(gbase, _G)],
                             wsem)
            return carry

        lax.fori_loop(0, _NCHUNKH, chunk, 0)
        drain_feat(0)
        drain_feat(1)

    return k(ta0, ta1, ta2, ta3, tb0, tb1, tb2, tb3, ma3, mb3)


_GBLK = 128
_BR = _GBLK * 128


def _mm_body(w_ref, ft_ref, *rest):
    o_ref = rest[-1]
    ft = ft_ref[...].reshape(8, _BR)
    o_ref[...] = lax.dot_general(
        w_ref[...], ft, (((0,), (0,)), ((), ())),
        preferred_element_type=jnp.float32)


_NBH = _R // 2 // _BR


def _mm_t_first(feat3, W):
    return pl.pallas_call(
        _mm_body,
        grid=(_NBH,),
        in_specs=[
            pl.BlockSpec((8, _OUT), lambda i: (0, 0)),
            pl.BlockSpec((8, _GBLK, 128), lambda i: (0, i, 0)),
        ],
        out_specs=pl.BlockSpec((_OUT, _BR), lambda i: (0, i)),
        out_shape=jax.ShapeDtypeStruct((_OUT, _R), jnp.float32),
    )(W, feat3)


def _mm_t_second(feat3, W, out_t):
    return pl.pallas_call(
        _mm_body,
        grid=(_NBH,),
        in_specs=[
            pl.BlockSpec((8, _OUT), lambda i: (0, 0)),
            pl.BlockSpec((8, _GBLK, 128), lambda i: (0, i, 0)),
            pl.BlockSpec(memory_space=pltpu.MemorySpace.HBM),
        ],
        out_specs=pl.BlockSpec((_OUT, _BR), lambda i: (0, i + _NBH)),
        out_shape=jax.ShapeDtypeStruct((_OUT, _R), jnp.float32),
        input_output_aliases={2: 0},
    )(W, feat3, out_t)


def kernel(codes, map_a, map_b, W):
    ca = codes[:_SIZES[0]].T
    cb = codes[_SIZES[0]:].T
    ma3 = map_a.T.reshape(_HDIM, _GP128, 128)
    mb3 = map_b.T.reshape(_HDIM, _GP128, 128)
    featA = _sc_gather(0, ca[0], ca[1], ca[2], ca[3],
                       cb[0], cb[1], cb[2], cb[3], ma3, mb3)
    featB = _sc_gather(1, ca[0], ca[1], ca[2], ca[3],
                       cb[0], cb[1], cb[2], cb[3], ma3, mb3)
    out_t = _mm_t_first(featA, W)
    out_t = _mm_t_second(featB, W, out_t)
    return out_t.T
